# Initial kernel scaffold; baseline (speedup 1.0000x reference)
import jax
import jax.numpy as jnp
from jax.experimental import pallas as pl


def kernel(x, edge_index, batch, W1, b1, W2, b2, Wc, bc):
    raise NotImplementedError("write your pallas kernel here")


# speedup vs baseline: 25.0533x; 25.0533x over previous
import functools

import jax
import jax.numpy as jnp
from jax import lax
from jax.experimental import pallas as pl
from jax.experimental.pallas import tpu as pltpu
from jax.experimental.pallas import tpu_sc as plsc

N = 100000
E = 1600000
IN_DIM = 3
HID = 64
G = 128

NC = 2
NS = 16
L = 16
NW = NC * NS

NP = 100352
EP = 1605632
CHUNK = 128
KQ = 4
RPS = NP // NS
GW1 = (EP // NW) // CHUNK
M1 = GW1 // KQ // 2
GW2 = (EP // NS) // CHUNK
M2 = GW2 // KQ // 2

_MESH = plsc.VectorSubcoreMesh(core_axis_name="c", subcore_axis_name="s")
_SC_PARAMS = pltpu.CompilerParams(use_tc_tiling_on_sc=False)


def _offadd(ebuf, off):
    if off is not None:
        for b in range(KQ):
            for k in range(CHUNK // L):
                ebuf[b, 0, pl.ds(k * L, L)] = ebuf[b, 0, pl.ds(k * L, L)] + off


def _fire_gather(table, ebuf, rows, sem):
    for b in range(KQ):
        pltpu.async_copy(
            table.at[ebuf.at[b, 0]], rows.at[pl.ds(b * CHUNK, CHUNK)], sem
        )


def _fire_scatter(rows, ebuf, acc, sem):
    for b in range(KQ):
        pltpu.async_copy(
            rows.at[pl.ds(b * CHUNK, CHUNK)], acc.at[ebuf.at[b, 1]], sem,
            add=True,
        )


def _drain_rows(table, rows, sem):
    for b in range(KQ):
        pltpu.make_async_copy(
            table.at[pl.ds(0, CHUNK)], rows.at[pl.ds(b * CHUNK, CHUNK)], sem
        ).wait()


def _idx_wait(edges2, ebuf, sem):
    pltpu.make_async_copy(edges2.at[pl.ds(0, KQ)], ebuf, sem).wait()


def _pipelined_pass(
    edges2, table, acc, base_row, npairs,
    ebuf_a, rows_a, gs_a, ss_a, ebuf_b, rows_b, gs_b, ss_b, is_b, off=None,
):
    pltpu.sync_copy(edges2.at[pl.ds(base_row, KQ)], ebuf_a)
    _offadd(ebuf_a, off)
    _fire_gather(table, ebuf_a, rows_a, gs_a)
    pltpu.async_copy(edges2.at[pl.ds(base_row + KQ, KQ)], ebuf_b, is_b)

    def pair(i, carry):
        _drain_rows(table, rows_a, gs_a)
        _fire_scatter(rows_a, ebuf_a, acc, ss_a)

        @pl.when(i > 0)
        def _free_b():
            _drain_rows(table, rows_b, ss_b)

        _idx_wait(edges2, ebuf_b, is_b)
        _offadd(ebuf_b, off)
        _fire_gather(table, ebuf_b, rows_b, gs_b)
        _drain_rows(table, rows_a, ss_a)

        @pl.when(i < npairs - 1)
        def _next_a():
            pltpu.sync_copy(
                edges2.at[pl.ds(base_row + (2 * i + 2) * KQ, KQ)], ebuf_a
            )
            _offadd(ebuf_a, off)
            _fire_gather(table, ebuf_a, rows_a, gs_a)
            pltpu.async_copy(
                edges2.at[pl.ds(base_row + (2 * i + 3) * KQ, KQ)], ebuf_b, is_b
            )

        _drain_rows(table, rows_b, gs_b)
        _fire_scatter(rows_b, ebuf_b, acc, ss_b)
        return carry

    lax.fori_loop(0, npairs, pair, 0)
    _drain_rows(table, rows_b, ss_b)


@functools.partial(
    pl.kernel,
    mesh=_MESH,
    out_type=jax.ShapeDtypeStruct((NC, NP), jnp.float32),
    scratch_types=[
        pltpu.VMEM((KQ, 2, CHUNK), jnp.int32),
        pltpu.VMEM((KQ, 2, CHUNK), jnp.int32),
        pltpu.VMEM((CHUNK,), jnp.float32),
        pltpu.VMEM_SHARED((NP,), jnp.float32),
        pltpu.SemaphoreType.DMA,
        pltpu.SemaphoreType.DMA,
        pltpu.SemaphoreType.DMA,
    ],
    compiler_params=_SC_PARAMS,
)
def _deg_kernel(edges2, ones_h, z1, out, ebuf_a, ebuf_b, onev, acc1,
                ss_a, ss_b, is_b):
    c = lax.axis_index("c")
    s = lax.axis_index("s")
    w = c * NS + s
    pltpu.sync_copy(z1.at[pl.ds(s * RPS, RPS)], acc1.at[pl.ds(s * RPS, RPS)])
    pltpu.sync_copy(ones_h, onev)
    plsc.subcore_barrier()
    base_row = w * GW1

    def fire(ebuf, sem):
        for b in range(KQ):
            pltpu.async_copy(onev, acc1.at[ebuf.at[b, 1]], sem, add=True)

    def drain(sem):
        for b in range(KQ):
            pltpu.make_async_copy(ones_h, onev, sem).wait()

    pltpu.sync_copy(edges2.at[pl.ds(base_row, KQ)], ebuf_a)
    fire(ebuf_a, ss_a)
    pltpu.async_copy(edges2.at[pl.ds(base_row + KQ, KQ)], ebuf_b, is_b)

    def pair(i, carry):
        @pl.when(i > 0)
        def _free_b():
            drain(ss_b)

        _idx_wait(edges2, ebuf_b, is_b)
        fire(ebuf_b, ss_b)
        drain(ss_a)

        @pl.when(i < M1 - 1)
        def _next_a():
            pltpu.sync_copy(
                edges2.at[pl.ds(base_row + (2 * i + 2) * KQ, KQ)], ebuf_a
            )
            fire(ebuf_a, ss_a)
            pltpu.async_copy(
                edges2.at[pl.ds(base_row + (2 * i + 3) * KQ, KQ)], ebuf_b, is_b
            )

        return carry

    lax.fori_loop(0, M1, pair, 0)
    drain(ss_b)
    plsc.subcore_barrier()
    pltpu.sync_copy(acc1.at[pl.ds(s * RPS, RPS)], out.at[c, pl.ds(s * RPS, RPS)])


_MSG_SCRATCH = (
    pltpu.VMEM((KQ, 2, CHUNK), jnp.int32),
    pltpu.VMEM((KQ * CHUNK, L), jnp.float32),
    pltpu.VMEM((KQ, 2, CHUNK), jnp.int32),
    pltpu.VMEM((KQ * CHUNK, L), jnp.float32),
    pltpu.VMEM_SHARED((NP, L), jnp.float32),
    pltpu.SemaphoreType.DMA,
    pltpu.SemaphoreType.DMA,
    pltpu.SemaphoreType.DMA,
    pltpu.SemaphoreType.DMA,
    pltpu.SemaphoreType.DMA,
)


@functools.partial(
    pl.kernel,
    mesh=_MESH,
    out_type=jax.ShapeDtypeStruct((NC, NP, L), jnp.float32),
    scratch_types=list(_MSG_SCRATCH),
    compiler_params=_SC_PARAMS,
)
def _msg1_kernel(
    edges2, table, z16, out,
    ebuf_a, rows_a, ebuf_b, rows_b, acc, gs_a, gs_b, ss_a, ss_b, is_b,
):
    c = lax.axis_index("c")
    s = lax.axis_index("s")
    w = c * NS + s
    pltpu.sync_copy(z16.at[pl.ds(s * RPS, RPS)], acc.at[pl.ds(s * RPS, RPS)])
    plsc.subcore_barrier()
    _pipelined_pass(
        edges2, table, acc, w * GW1, M1,
        ebuf_a, rows_a, gs_a, ss_a, ebuf_b, rows_b, gs_b, ss_b, is_b,
    )
    plsc.subcore_barrier()
    pltpu.sync_copy(
        acc.at[pl.ds(s * RPS, RPS)], out.at[c, pl.ds(s * RPS, RPS)]
    )


@functools.partial(
    pl.kernel,
    mesh=_MESH,
    out_type=jax.ShapeDtypeStruct((4, NP, L), jnp.float32),
    scratch_types=list(_MSG_SCRATCH),
    compiler_params=_SC_PARAMS,
)
def _msg2_kernel(
    edges2, table4, z16, out,
    ebuf_a, rows_a, ebuf_b, rows_b, acc, gs_a, gs_b, ss_a, ss_b, is_b,
):
    c = lax.axis_index("c")
    s = lax.axis_index("s")
    for cc in range(2):
        chunk = c * 2 + cc
        off = chunk * NP
        pltpu.sync_copy(z16.at[pl.ds(s * RPS, RPS)], acc.at[pl.ds(s * RPS, RPS)])
        plsc.subcore_barrier()
        _pipelined_pass(
            edges2, table4, acc, s * GW2, M2,
            ebuf_a, rows_a, gs_a, ss_a, ebuf_b, rows_b, gs_b, ss_b, is_b,
            off=off,
        )
        plsc.subcore_barrier()
        pltpu.sync_copy(
            acc.at[pl.ds(s * RPS, RPS)], out.at[chunk, pl.ds(s * RPS, RPS)]
        )
        plsc.subcore_barrier()


BR = 3584
NG = NP // BR


def _norm_body(dega, degb, xp, dinv_o, u_o):
    deg = dega[...] + degb[...] + 1.0
    dv = lax.rsqrt(deg)
    dinv_o[...] = dv
    u_o[...] = dv * xp[...]


def _norm_call(dega, degb, xp):
    return pl.pallas_call(
        _norm_body,
        grid=(NG,),
        in_specs=[
            pl.BlockSpec((BR, 1), lambda i: (i, 0)),
            pl.BlockSpec((BR, 1), lambda i: (i, 0)),
            pl.BlockSpec((BR, L), lambda i: (i, 0)),
        ],
        out_specs=[
            pl.BlockSpec((BR, 1), lambda i: (i, 0)),
            pl.BlockSpec((BR, L), lambda i: (i, 0)),
        ],
        out_shape=[
            jax.ShapeDtypeStruct((NP, 1), jnp.float32),
            jax.ShapeDtypeStruct((NP, L), jnp.float32),
        ],
    )(dega, degb, xp)


def _dense_body(s0a, s0b, xp, dinv, w1p, b1r, w2, g2_o, hw_o):
    dv = dinv[...]
    t = dv * (s0a[...] + s0b[...]) + dv * dv * xp[...]
    h1 = jnp.maximum(
        jax.lax.dot(t, w1p[...], preferred_element_type=jnp.float32) + b1r[...],
        0.0,
    )
    hw = jax.lax.dot(h1, w2[...], preferred_element_type=jnp.float32)
    hw_o[...] = hw
    g2_o[...] = dv * hw


def _dense_call(s0a, s0b, xp, dinv, w1p, b1r, w2):
    return pl.pallas_call(
        _dense_body,
        grid=(NG,),
        in_specs=[
            pl.BlockSpec((BR, L), lambda i: (i, 0)),
            pl.BlockSpec((BR, L), lambda i: (i, 0)),
            pl.BlockSpec((BR, L), lambda i: (i, 0)),
            pl.BlockSpec((BR, 1), lambda i: (i, 0)),
            pl.BlockSpec((L, HID), lambda i: (0, 0)),
            pl.BlockSpec((1, HID), lambda i: (0, 0)),
            pl.BlockSpec((HID, HID), lambda i: (0, 0)),
        ],
        out_specs=[
            pl.BlockSpec((BR, HID), lambda i: (i, 0)),
            pl.BlockSpec((BR, HID), lambda i: (i, 0)),
        ],
        out_shape=[
            jax.ShapeDtypeStruct((NP, HID), jnp.float32),
            jax.ShapeDtypeStruct((NP, HID), jnp.float32),
        ],
    )(s0a, s0b, xp, dinv, w1p, b1r, w2)


def _pool_body(s2, hw, dinv, b2r, bat, wc, bcr, sums_o, cnt_o, res_o):
    i = pl.program_id(0)

    @pl.when(i == 0)
    def _init():
        sums_o[...] = jnp.zeros_like(sums_o)
        cnt_o[...] = jnp.zeros_like(cnt_o)

    dv = dinv[...]
    h2 = jnp.maximum(dv * s2[...] + dv * dv * hw[...] + b2r[...], 0.0)
    ids = lax.broadcasted_iota(jnp.int32, (BR, G), 1)
    oh = (bat[...] == ids).astype(jnp.float32)
    sums_o[...] += jax.lax.dot_general(
        oh, h2, (((0,), (0,)), ((), ())), preferred_element_type=jnp.float32
    )
    cnt_o[...] += jax.lax.dot_general(
        oh,
        jnp.ones((BR, 1), jnp.float32),
        (((0,), (0,)), ((), ())),
        preferred_element_type=jnp.float32,
    )

    @pl.when(i == NG - 1)
    def _fin():
        pooled = sums_o[...] / jnp.maximum(cnt_o[...], 1.0)
        z = jax.lax.dot(pooled, wc[...], preferred_element_type=jnp.float32)
        res_o[...] = jax.nn.sigmoid(z + bcr[...])


def _pool_call(s2, hw, dinv, b2r, bat2, wc, bcr):
    return pl.pallas_call(
        _pool_body,
        grid=(NG,),
        in_specs=[
            pl.BlockSpec((BR, HID), lambda i: (i, 0)),
            pl.BlockSpec((BR, HID), lambda i: (i, 0)),
            pl.BlockSpec((BR, 1), lambda i: (i, 0)),
            pl.BlockSpec((1, HID), lambda i: (0, 0)),
            pl.BlockSpec((BR, 1), lambda i: (i, 0)),
            pl.BlockSpec((HID, 1), lambda i: (0, 0)),
            pl.BlockSpec((1, 1), lambda i: (0, 0)),
        ],
        out_specs=[
            pl.BlockSpec((G, HID), lambda i: (0, 0)),
            pl.BlockSpec((G, 1), lambda i: (0, 0)),
            pl.BlockSpec((G, 1), lambda i: (0, 0)),
        ],
        out_shape=[
            jax.ShapeDtypeStruct((G, HID), jnp.float32),
            jax.ShapeDtypeStruct((G, 1), jnp.float32),
            jax.ShapeDtypeStruct((G, 1), jnp.float32),
        ],
    )(s2, hw, dinv, b2r, bat2, wc, bcr)


def kernel(x, edge_index, batch, W1, b1, W2, b2, Wc, bc):
    src = edge_index[0].astype(jnp.int32)
    dst = edge_index[1].astype(jnp.int32)
    epad = jnp.full((EP - E,), NP - 1, jnp.int32)
    src2 = jnp.concatenate([src, epad]).reshape(EP // CHUNK, CHUNK)
    dst2 = jnp.concatenate([dst, epad]).reshape(EP // CHUNK, CHUNK)
    edges2 = jnp.stack([src2, dst2], axis=1)
    xp = jnp.pad(x, ((0, NP - N), (0, L - IN_DIM)))
    bat2 = jnp.pad(
        batch.astype(jnp.int32), (0, NP - N), constant_values=G
    ).reshape(NP, 1)
    z16 = jnp.zeros((NP, L), jnp.float32)
    z1 = jnp.zeros((NP,), jnp.float32)
    onesc = jnp.ones((CHUNK,), jnp.float32)
    w1p = jnp.pad(W1, ((0, L - IN_DIM), (0, 0)))
    b1r = b1.reshape(1, HID)
    b2r = b2.reshape(1, HID)
    bcr = bc.reshape(1, 1)

    degp = _deg_kernel(edges2, onesc, z1)
    dinv, U = _norm_call(
        degp[0].reshape(NP, 1), degp[1].reshape(NP, 1), xp
    )
    s0 = _msg1_kernel(edges2, U, z16)
    g2, hw = _dense_call(s0[0], s0[1], xp, dinv, w1p, b1r, W2)
    g2t = g2.reshape(NP, 4, L).transpose(1, 0, 2).reshape(4 * NP, L)
    s2 = _msg2_kernel(edges2, g2t, z16)
    s2cat = s2.transpose(1, 0, 2).reshape(NP, HID)
    _, _, res = _pool_call(s2cat, hw, dinv, b2r, bat2, Wc, bcr)
    return res.reshape(G)
